# Initial kernel scaffold; baseline (speedup 1.0000x reference)
import jax
import jax.numpy as jnp
from jax.experimental import pallas as pl


def kernel(x, index):
    raise NotImplementedError("write your pallas kernel here")


# speedup vs baseline: 10.9261x; 10.9261x over previous
import functools

import jax
import jax.numpy as jnp
from jax import lax
from jax.experimental import pallas as pl
from jax.experimental.pallas import tpu as pltpu
from jax.experimental.pallas import tpu_sc as plsc

N = 1600000
S = 100000
NC = 2
NS = 16
NW = NC * NS
C = N // NW
T = 10000
L = 16
VPT = T // L
TILES = C // T
K = 128
THRESH = K - L
DUMP = K - 1
MAXI = jnp.iinfo(jnp.int32).max


def _body(x_hbm, idx_hbm, out_hbm, x_buf, idx_buf, stage_seg, stage_val):
  iota = lax.iota(jnp.int32, L)

  def splat_i(v):
    return jnp.full((L,), v, jnp.int32)

  def splat_f(v):
    return jnp.full((L,), v, jnp.float32)

  def lane(v, l):
    return jnp.take_along_axis(v, splat_i(l), axis=0, mode="promise_in_bounds")

  def shift(v, d):
    return jnp.take_along_axis(
        v, jnp.maximum(iota - d, 0), axis=0, mode="promise_in_bounds")

  core = lax.axis_index("c")
  sub = lax.axis_index("s")
  wid = (sub * NC + core).astype(jnp.int32)
  chunk_start = wid * C
  chunk_end = chunk_start + C

  def reset_stage():
    for j in range(K // L):
      stage_seg[pl.ds(j * L, L)] = splat_i(S)

  def flush():
    pltpu.sync_copy(stage_val, out_hbm.at[stage_seg])
    reset_stage()

  def mask_prefix(mask):
    m_i = jnp.where(mask, jnp.int32(1), jnp.int32(0))
    incl = m_i
    for d in (1, 2, 4, 8):
      sh = jnp.take_along_axis(
          incl, jnp.maximum(iota - d, 0), axis=0, mode="promise_in_bounds")
      incl = incl + jnp.where(iota >= d, sh, jnp.int32(0))
    return incl - m_i, lane(incl, 15)[0]

  def append(cursor, segs, vals, mask):
    offs, cnt = mask_prefix(mask)
    dest = jnp.where(mask, splat_i(cursor) + offs, splat_i(DUMP))
    plsc.store_scatter(stage_seg, [dest], jnp.where(mask, segs, splat_i(S)))
    plsc.store_scatter(stage_val, [dest], vals)
    cursor = cursor + cnt
    need = cursor > THRESH
    pl.when(need)(flush)
    return jnp.where(need, jnp.int32(0), cursor)

  def gap_fill(cursor, a, b):
    nit = lax.div(b - a + (L - 1), jnp.int32(L))

    def body(i, cur):
      t = a + i * L
      segs = splat_i(t) + iota
      m = segs < splat_i(b)
      return append(cur, segs, splat_i(MAXI), m)

    return lax.fori_loop(0, nit, body, cursor)

  def process_vreg(st, o, p):
    cursor, c_seg, c_mval, c_mpos, c_rstart = st[:5]
    xv = x_buf[pl.ds(o, L)]
    sv = idx_buf[pl.ds(o, L)]
    nseg = plsc.load_gather(idx_buf, [splat_i(o) + iota + 1])
    pos = splat_i(p) + iota
    mval, mpos, rstart = xv, pos, pos
    for d in (1, 2, 4, 8):
      pseg = shift(sv, d)
      pmv = shift(mval, d)
      pmp = shift(mpos, d)
      prs = shift(rstart, d)
      valid = (iota >= d) & (pseg == sv)
      take = valid & ((pmv > mval) | ((pmv == mval) & (pmp < mpos)))
      mval = jnp.where(take, pmv, mval)
      mpos = jnp.where(take, pmp, mpos)
      rstart = jnp.where(valid, jnp.minimum(prs, rstart), rstart)
    s0 = lane(sv, 0)
    fm = (sv == s0) & (c_seg == s0)
    take = fm & ((c_mval > mval) | ((c_mval == mval) & (c_mpos < mpos)))
    mval = jnp.where(take, c_mval, mval)
    mpos = jnp.where(take, c_mpos, mpos)
    rstart = jnp.where(fm, jnp.minimum(rstart, c_rstart), rstart)
    close = sv != nseg
    owned = (rstart >= splat_i(chunk_start)) & (rstart < splat_i(chunk_end))
    emit = close & owned
    cursor = append(cursor, sv, mpos - rstart, emit)
    gmask = emit & (nseg > sv + 1)
    gm_i = jnp.where(gmask, jnp.int32(1), jnp.int32(0))
    _, has_gap = mask_prefix(gmask)

    def do_gaps(cur):
      def lane_body(l, cur):
        g = lane(gm_i, l)[0]
        a = lane(sv, l)[0] + 1
        b = lane(nseg, l)[0]
        return lax.cond(g > 0, lambda c: gap_fill(c, a, b), lambda c: c, cur)

      return lax.fori_loop(0, L, lane_body, cur)

    cursor = lax.cond(has_gap > 0, do_gaps, lambda c: c, cursor)
    open_run = lane(jnp.where(close, jnp.int32(1), jnp.int32(0)), 15) == 0
    c_seg = jnp.where(open_run, lane(sv, 15), splat_i(-1))
    c_mval = jnp.where(open_run, lane(mval, 15), splat_f(0.0))
    c_mpos = jnp.where(open_run, lane(mpos, 15), splat_i(0))
    c_rstart = jnp.where(open_run, lane(rstart, 15), splat_i(N))
    return (cursor, c_seg, c_mval, c_mpos, c_rstart, c_rstart[0])

  reset_stage()

  @pl.when(wid > 0)
  def _():
    off = pl.multiple_of(chunk_start - L, 16)
    pltpu.sync_copy(idx_hbm.at[pl.ds(off, L)], idx_buf.at[pl.ds(0, L)])

  wv = jnp.full((L,), wid, jnp.int32)
  prev = lane(idx_buf[pl.ds(0, L)], 15)
  c_seg0 = jnp.where(wv > 0, prev, splat_i(-1))
  state = (jnp.int32(0), c_seg0, splat_f(-jnp.inf), splat_i(0),
           splat_i(chunk_start - 1), chunk_start - 1)

  def tile_body(t, st):
    base = pl.multiple_of(chunk_start + t * T, 16)
    pltpu.sync_copy(x_hbm.at[pl.ds(base, T)], x_buf.at[pl.ds(0, T)])
    pltpu.sync_copy(idx_hbm.at[pl.ds(base, T)], idx_buf.at[pl.ds(0, T)])

    @pl.when(base + T < N)
    def _():
      off = pl.multiple_of(base + T, 16)
      pltpu.sync_copy(idx_hbm.at[pl.ds(off, L)], idx_buf.at[pl.ds(T, L)])

    @pl.when(base + T == N)
    def _():
      idx_buf[pl.ds(T, L)] = splat_i(S)

    def lead(stt):
      first = idx_buf[pl.ds(0, L)][0]
      cur = gap_fill(stt[0], jnp.int32(0), first)
      return (cur,) + stt[1:]

    st = lax.cond((t == 0) & (wid == 0), lead, lambda z: z, st)

    def vreg_body(v, stt):
      return process_vreg(stt, v * L, base + v * L)

    return lax.fori_loop(0, VPT, vreg_body, st)

  state = lax.fori_loop(0, TILES, tile_body, state)

  def cond2(st2):
    rs = st2[5]
    return (rs >= chunk_start) & (rs < chunk_end)

  def body2(st2):
    st, base = st2[:6], st2[6]
    base = pl.multiple_of(base, 16)
    pltpu.sync_copy(x_hbm.at[pl.ds(base, L)], x_buf.at[pl.ds(0, L)])
    pltpu.sync_copy(idx_hbm.at[pl.ds(base, L)], idx_buf.at[pl.ds(0, L)])

    @pl.when(base + L < N)
    def _():
      off = pl.multiple_of(base + L, 16)
      pltpu.sync_copy(idx_hbm.at[pl.ds(off, L)], idx_buf.at[pl.ds(L, L)])

    @pl.when(base + L == N)
    def _():
      idx_buf[pl.ds(L, L)] = splat_i(S)

    st = process_vreg(st, 0, base)
    return st + (base + L,)

  st2 = lax.while_loop(cond2, body2, state + (chunk_end.astype(jnp.int32),))
  del st2
  flush()


@jax.jit
def _run(x, idx32):
  mesh = plsc.VectorSubcoreMesh(
      core_axis_name="c", subcore_axis_name="s", num_cores=NC, num_subcores=NS)
  out = pl.kernel(
      _body,
      out_type=jax.ShapeDtypeStruct((S + L,), jnp.int32),
      mesh=mesh,
      compiler_params=pltpu.CompilerParams(needs_layout_passes=False),
      scratch_types=[
          pltpu.VMEM((T,), jnp.float32),
          pltpu.VMEM((T + L,), jnp.int32),
          pltpu.VMEM((K,), jnp.int32),
          pltpu.VMEM((K,), jnp.int32),
      ],
  )(x, idx32)
  return out[:S]


def kernel(x, index):
  return _run(x, index.astype(jnp.int32))


# speedup vs baseline: 10.9281x; 1.0002x over previous
import functools

import jax
import jax.numpy as jnp
from jax import lax
from jax.experimental import pallas as pl
from jax.experimental.pallas import tpu as pltpu
from jax.experimental.pallas import tpu_sc as plsc

N = 1600000
S = 100000
NC = 2
NS = 16
NW = NC * NS
C = N // NW
T = 50000
L = 16
VPT = T // L
TILES = C // T
K = 128
THRESH = K - L
DUMP = K - 1
MAXI = jnp.iinfo(jnp.int32).max


def _body(x_hbm, idx_hbm, out_hbm, x_buf, idx_buf, stage_seg, stage_val):
  iota = lax.iota(jnp.int32, L)

  def splat_i(v):
    return jnp.full((L,), v, jnp.int32)

  def splat_f(v):
    return jnp.full((L,), v, jnp.float32)

  def lane(v, l):
    return jnp.take_along_axis(v, splat_i(l), axis=0, mode="promise_in_bounds")

  def shift(v, d):
    return jnp.take_along_axis(
        v, jnp.maximum(iota - d, 0), axis=0, mode="promise_in_bounds")

  core = lax.axis_index("c")
  sub = lax.axis_index("s")
  wid = (sub * NC + core).astype(jnp.int32)
  chunk_start = wid * C
  chunk_end = chunk_start + C

  def reset_stage():
    for j in range(K // L):
      stage_seg[pl.ds(j * L, L)] = splat_i(S)

  def flush():
    pltpu.sync_copy(stage_val, out_hbm.at[stage_seg])
    reset_stage()

  def mask_prefix(mask):
    m_i = jnp.where(mask, jnp.int32(1), jnp.int32(0))
    incl = m_i
    for d in (1, 2, 4, 8):
      sh = jnp.take_along_axis(
          incl, jnp.maximum(iota - d, 0), axis=0, mode="promise_in_bounds")
      incl = incl + jnp.where(iota >= d, sh, jnp.int32(0))
    return incl - m_i, lane(incl, 15)[0]

  def append(cursor, segs, vals, mask):
    offs, cnt = mask_prefix(mask)
    dest = jnp.where(mask, splat_i(cursor) + offs, splat_i(DUMP))
    plsc.store_scatter(stage_seg, [dest], jnp.where(mask, segs, splat_i(S)))
    plsc.store_scatter(stage_val, [dest], vals)
    cursor = cursor + cnt
    need = cursor > THRESH
    pl.when(need)(flush)
    return jnp.where(need, jnp.int32(0), cursor)

  def gap_fill(cursor, a, b):
    nit = lax.div(b - a + (L - 1), jnp.int32(L))

    def body(i, cur):
      t = a + i * L
      segs = splat_i(t) + iota
      m = segs < splat_i(b)
      return append(cur, segs, splat_i(MAXI), m)

    return lax.fori_loop(0, nit, body, cursor)

  def process_vreg(st, o, p):
    cursor, c_seg, c_mval, c_mpos, c_rstart = st[:5]
    xv = x_buf[pl.ds(o, L)]
    sv = idx_buf[pl.ds(o, L)]
    nseg = plsc.load_gather(idx_buf, [splat_i(o) + iota + 1])
    pos = splat_i(p) + iota
    mval, mpos, rstart = xv, pos, pos
    for d in (1, 2, 4, 8):
      pseg = shift(sv, d)
      pmv = shift(mval, d)
      pmp = shift(mpos, d)
      prs = shift(rstart, d)
      valid = (iota >= d) & (pseg == sv)
      take = valid & ((pmv > mval) | ((pmv == mval) & (pmp < mpos)))
      mval = jnp.where(take, pmv, mval)
      mpos = jnp.where(take, pmp, mpos)
      rstart = jnp.where(valid, jnp.minimum(prs, rstart), rstart)
    s0 = lane(sv, 0)
    fm = (sv == s0) & (c_seg == s0)
    take = fm & ((c_mval > mval) | ((c_mval == mval) & (c_mpos < mpos)))
    mval = jnp.where(take, c_mval, mval)
    mpos = jnp.where(take, c_mpos, mpos)
    rstart = jnp.where(fm, jnp.minimum(rstart, c_rstart), rstart)
    close = sv != nseg
    owned = (rstart >= splat_i(chunk_start)) & (rstart < splat_i(chunk_end))
    emit = close & owned
    cursor = append(cursor, sv, mpos - rstart, emit)
    gmask = emit & (nseg > sv + 1)
    gm_i = jnp.where(gmask, jnp.int32(1), jnp.int32(0))
    _, has_gap = mask_prefix(gmask)

    def do_gaps(cur):
      def lane_body(l, cur):
        g = lane(gm_i, l)[0]
        a = lane(sv, l)[0] + 1
        b = lane(nseg, l)[0]
        return lax.cond(g > 0, lambda c: gap_fill(c, a, b), lambda c: c, cur)

      return lax.fori_loop(0, L, lane_body, cur)

    cursor = lax.cond(has_gap > 0, do_gaps, lambda c: c, cursor)
    open_run = lane(jnp.where(close, jnp.int32(1), jnp.int32(0)), 15) == 0
    c_seg = jnp.where(open_run, lane(sv, 15), splat_i(-1))
    c_mval = jnp.where(open_run, lane(mval, 15), splat_f(0.0))
    c_mpos = jnp.where(open_run, lane(mpos, 15), splat_i(0))
    c_rstart = jnp.where(open_run, lane(rstart, 15), splat_i(N))
    return (cursor, c_seg, c_mval, c_mpos, c_rstart, c_rstart[0])

  reset_stage()

  @pl.when(wid > 0)
  def _():
    off = pl.multiple_of(chunk_start - L, 16)
    pltpu.sync_copy(idx_hbm.at[pl.ds(off, L)], idx_buf.at[pl.ds(0, L)])

  wv = jnp.full((L,), wid, jnp.int32)
  prev = lane(idx_buf[pl.ds(0, L)], 15)
  c_seg0 = jnp.where(wv > 0, prev, splat_i(-1))
  state = (jnp.int32(0), c_seg0, splat_f(-jnp.inf), splat_i(0),
           splat_i(chunk_start - 1), chunk_start - 1)

  def tile_body(t, st):
    base = pl.multiple_of(chunk_start + t * T, 16)
    pltpu.sync_copy(x_hbm.at[pl.ds(base, T)], x_buf.at[pl.ds(0, T)])
    pltpu.sync_copy(idx_hbm.at[pl.ds(base, T)], idx_buf.at[pl.ds(0, T)])

    @pl.when(base + T < N)
    def _():
      off = pl.multiple_of(base + T, 16)
      pltpu.sync_copy(idx_hbm.at[pl.ds(off, L)], idx_buf.at[pl.ds(T, L)])

    @pl.when(base + T == N)
    def _():
      idx_buf[pl.ds(T, L)] = splat_i(S)

    def lead(stt):
      first = idx_buf[pl.ds(0, L)][0]
      cur = gap_fill(stt[0], jnp.int32(0), first)
      return (cur,) + stt[1:]

    st = lax.cond((t == 0) & (wid == 0), lead, lambda z: z, st)

    def vreg_body(v, stt):
      return process_vreg(stt, v * L, base + v * L)

    return lax.fori_loop(0, VPT, vreg_body, st)

  state = lax.fori_loop(0, TILES, tile_body, state)

  def cond2(st2):
    rs = st2[5]
    return (rs >= chunk_start) & (rs < chunk_end)

  def body2(st2):
    st, base = st2[:6], st2[6]
    base = pl.multiple_of(base, 16)
    pltpu.sync_copy(x_hbm.at[pl.ds(base, L)], x_buf.at[pl.ds(0, L)])
    pltpu.sync_copy(idx_hbm.at[pl.ds(base, L)], idx_buf.at[pl.ds(0, L)])

    @pl.when(base + L < N)
    def _():
      off = pl.multiple_of(base + L, 16)
      pltpu.sync_copy(idx_hbm.at[pl.ds(off, L)], idx_buf.at[pl.ds(L, L)])

    @pl.when(base + L == N)
    def _():
      idx_buf[pl.ds(L, L)] = splat_i(S)

    st = process_vreg(st, 0, base)
    return st + (base + L,)

  st2 = lax.while_loop(cond2, body2, state + (chunk_end.astype(jnp.int32),))
  del st2
  flush()


@jax.jit
def _run(x, idx32):
  mesh = plsc.VectorSubcoreMesh(
      core_axis_name="c", subcore_axis_name="s", num_cores=NC, num_subcores=NS)
  out = pl.kernel(
      _body,
      out_type=jax.ShapeDtypeStruct((S + L,), jnp.int32),
      mesh=mesh,
      compiler_params=pltpu.CompilerParams(needs_layout_passes=False),
      scratch_types=[
          pltpu.VMEM((T,), jnp.float32),
          pltpu.VMEM((T + L,), jnp.int32),
          pltpu.VMEM((K,), jnp.int32),
          pltpu.VMEM((K,), jnp.int32),
      ],
  )(x, idx32)
  return out[:S]


def kernel(x, index):
  return _run(x, index.astype(jnp.int32))
